# Initial kernel scaffold; baseline (speedup 1.0000x reference)
import jax
import jax.numpy as jnp
from jax.experimental import pallas as pl


def kernel(graph, pos, z_0, z_1, emb, edgelabels, W_label, b_label, W_src, b_src, W_dst, b_dst, W_gate, b_gate):
    raise NotImplementedError("write your pallas kernel here")


# speedup vs baseline: 14.4123x; 14.4123x over previous
import functools

import jax
import jax.numpy as jnp
from jax import lax
from jax.experimental import pallas as pl
from jax.experimental.pallas import tpu as pltpu
from jax.experimental.pallas import tpu_sc as plsc

NC = 2
NS = 16
NW = NC * NS
CK = 80


def _node_proj_body(emb_ref, pos_ref, wsrc_ref, bsrc_ref, wdst_ref, bdst_ref,
                    p_ref, q_ref):
    chan = emb_ref.shape[1]
    e = emb_ref[...]
    blk = e.shape[0]
    posb = pos_ref[...]
    pad = jnp.zeros((blk, chan - posb.shape[1]), jnp.float32)
    p_ref[...] = jnp.concatenate([
        jnp.dot(e, wsrc_ref[...], preferred_element_type=jnp.float32)
        + bsrc_ref[...], -0.1 * posb, pad], axis=1)
    q_ref[...] = jnp.concatenate([
        jnp.dot(e, wdst_ref[...], preferred_element_type=jnp.float32)
        + bdst_ref[...], 0.1 * posb, pad], axis=1)


def _node_proj(emb, pos, w_src, b_src, w_dst, b_dst, blk):
    n, chan = emb.shape
    pc = pos.shape[1]
    grid = n // blk
    full = lambda i: (0, 0)
    return pl.pallas_call(
        _node_proj_body,
        grid=(grid,),
        in_specs=[
            pl.BlockSpec((blk, chan), lambda i: (i, 0)),
            pl.BlockSpec((blk, pc), lambda i: (i, 0)),
            pl.BlockSpec((chan, chan), full),
            pl.BlockSpec((1, chan), full),
            pl.BlockSpec((chan, chan), full),
            pl.BlockSpec((1, chan), full),
        ],
        out_specs=[
            pl.BlockSpec((blk, 2 * chan), lambda i: (i, 0)),
            pl.BlockSpec((blk, 2 * chan), lambda i: (i, 0)),
        ],
        out_shape=[
            jax.ShapeDtypeStruct((n, 2 * chan), jnp.float32),
            jax.ShapeDtypeStruct((n, 2 * chan), jnp.float32),
        ],
    )(emb, pos, w_src, b_src.reshape(1, chan), w_dst, b_dst.reshape(1, chan))


def _sc_gather_body(src_hbm, dst_hbm, p_hbm, q_hbm, z01_hbm,
                    s_hbm, zg_hbm,
                    srcv, dstv, pgv, qgv, zgv,
                    s0, s1, s2):
    cid = lax.axis_index("c")
    sid = lax.axis_index("s")
    wid = cid * NS + sid
    epw = src_hbm.shape[0] // NW
    base = wid * epw
    nch = epw // CK
    nv = pgv.shape[1] // 16

    def chunk(t, carry):
        off = base + t * CK
        pltpu.sync_copy(src_hbm.at[pl.ds(off, CK)], srcv)
        pltpu.sync_copy(dst_hbm.at[pl.ds(off, CK)], dstv)
        d1 = pltpu.async_copy(p_hbm.at[srcv], pgv, s0)
        d2 = pltpu.async_copy(q_hbm.at[dstv], qgv, s1)
        d3 = pltpu.async_copy(z01_hbm.at[dstv], zgv, s2)
        d1.wait()
        d2.wait()

        def abody(e, c):
            for j in range(nv):
                s = pl.ds(j * 16, 16)
                pgv[e, s] = pgv[e, s] + qgv[e, s]
            return c
        lax.fori_loop(0, CK, abody, 0)

        d3.wait()
        w1 = pltpu.async_copy(pgv, s_hbm.at[pl.ds(off, CK)], s0)
        w2 = pltpu.async_copy(zgv, zg_hbm.at[pl.ds(off, CK)], s1)
        w1.wait()
        w2.wait()
        return carry

    lax.fori_loop(0, nch, chunk, 0)


def _sc_gather(src, dst, p, q, z01):
    e = src.shape[0]
    pw = p.shape[1]
    zc = z01.shape[1]
    mesh = plsc.VectorSubcoreMesh(core_axis_name="c", subcore_axis_name="s",
                                  num_cores=NC, num_subcores=NS)
    f = functools.partial(
        pl.kernel,
        out_type=[
            jax.ShapeDtypeStruct((e, pw), jnp.float32),
            jax.ShapeDtypeStruct((e, zc), jnp.float32),
        ],
        mesh=mesh,
        scratch_types=[
            pltpu.VMEM((CK,), jnp.int32),
            pltpu.VMEM((CK,), jnp.int32),
            pltpu.VMEM((CK, pw), jnp.float32),
            pltpu.VMEM((CK, pw), jnp.float32),
            pltpu.VMEM((CK, zc), jnp.float32),
            pltpu.SemaphoreType.DMA,
            pltpu.SemaphoreType.DMA,
            pltpu.SemaphoreType.DMA,
        ],
    )(_sc_gather_body)
    return f(src, dst, p, q, z01)


def _tc_gate_body(s_ref, lab_ref, zg_ref, wl_ref, bl_ref, wg_ref,
                  bg_ref, p0_ref, p1_ref, p2_ref, p3_ref):
    chan = wl_ref.shape[1]
    sv = s_ref[...]
    a = (sv[:, :chan]
         + jnp.dot(lab_ref[...], wl_ref[...], preferred_element_type=jnp.float32)
         + bl_ref[...])
    r = sv[:, chan:chan + 3]
    s = a * jax.nn.sigmoid(a)
    g = jnp.dot(s, wg_ref[...], preferred_element_type=jnp.float32) + bg_ref[...]
    g0 = g[:, :chan]
    g1 = g[:, chan:2 * chan]
    g2 = g[:, 2 * chan:]
    zg = zg_ref[...]
    p0_ref[...] = g0 * zg[:, :chan]
    p1_ref[...] = g1 * zg[:, chan:2 * chan] + g2 * r[:, 0:1]
    p2_ref[...] = g1 * zg[:, 2 * chan:3 * chan] + g2 * r[:, 1:2]
    p3_ref[...] = g1 * zg[:, 3 * chan:] + g2 * r[:, 2:3]


def _tc_gates(s, lab, zg, w_label, b_label, w_gate, b_gate, blk):
    e, pw = s.shape
    ed, chan = w_label.shape
    zc = zg.shape[1]
    grid = e // blk
    full = lambda i: (0, 0)
    row = lambda i: (i, 0)
    outs = [jax.ShapeDtypeStruct((e, chan), jnp.float32) for _ in range(4)]
    return pl.pallas_call(
        _tc_gate_body,
        grid=(grid,),
        in_specs=[
            pl.BlockSpec((blk, pw), row),
            pl.BlockSpec((blk, ed), row),
            pl.BlockSpec((blk, zc), row),
            pl.BlockSpec((ed, chan), full),
            pl.BlockSpec((1, chan), full),
            pl.BlockSpec((chan, 3 * chan), full),
            pl.BlockSpec((1, 3 * chan), full),
        ],
        out_specs=[pl.BlockSpec((blk, chan), row) for _ in range(4)],
        out_shape=outs,
    )(s, lab, zg, w_label, b_label.reshape(1, chan), w_gate,
      b_gate.reshape(1, 3 * chan))


def _sc_scatter_body(src_hbm, p0_hbm, p1_hbm, p2_hbm, p3_hbm, zeros_hbm,
                     o0_hbm, o1_hbm, o2_hbm, o3_hbm,
                     idxv, updv, acc):
    cid = lax.axis_index("c")
    sid = lax.axis_index("s")
    e = src_hbm.shape[0]
    n = zeros_hbm.shape[0]
    epc = e // NS
    nch = epc // CK
    rps = (n // NS) & ~7
    tail = n - NS * rps

    def do_group(psi_hbm, out_hbm):
        rows = pl.ds(sid * rps, rps)
        trows = pl.ds(NS * rps, tail)
        pltpu.sync_copy(zeros_hbm.at[rows], acc.at[rows])

        @pl.when(sid == NS - 1)
        def _():
            pltpu.sync_copy(zeros_hbm.at[trows], acc.at[trows])

        plsc.subcore_barrier()

        def chunk(t, carry):
            off = sid * epc + t * CK
            pltpu.sync_copy(src_hbm.at[pl.ds(off, CK)], idxv)
            pltpu.sync_copy(psi_hbm.at[pl.ds(off, CK)], updv)
            pltpu.sync_copy(updv, acc.at[idxv], add=True)
            return carry

        lax.fori_loop(0, nch, chunk, 0)
        plsc.subcore_barrier()
        pltpu.sync_copy(acc.at[rows], out_hbm.at[rows])

        @pl.when(sid == NS - 1)
        def _():
            pltpu.sync_copy(acc.at[trows], out_hbm.at[trows])

        plsc.subcore_barrier()

    @pl.when(cid == 0)
    def _():
        do_group(p0_hbm, o0_hbm)
        do_group(p1_hbm, o1_hbm)

    @pl.when(cid == 1)
    def _():
        do_group(p2_hbm, o2_hbm)
        do_group(p3_hbm, o3_hbm)


def _sc_scatter(src, p0, p1, p2, p3, zeros):
    n, chan = zeros.shape
    mesh = plsc.VectorSubcoreMesh(core_axis_name="c", subcore_axis_name="s",
                                  num_cores=NC, num_subcores=NS)
    out = jax.ShapeDtypeStruct((n, chan), jnp.float32)
    f = functools.partial(
        pl.kernel,
        out_type=[out, out, out, out],
        mesh=mesh,
        scratch_types=[
            pltpu.VMEM((CK,), jnp.int32),
            pltpu.VMEM((CK, chan), jnp.float32),
            pltpu.VMEM_SHARED((n, chan), jnp.float32),
        ],
    )(_sc_scatter_body)
    return f(src, p0, p1, p2, p3, zeros)


def kernel(graph, pos, z_0, z_1, emb, edgelabels,
           W_label, b_label, W_src, b_src, W_dst, b_dst, W_gate, b_gate):
    n, chan = z_0.shape
    src = graph[0]
    dst = graph[1]

    px, qx = _node_proj(emb, pos, W_src, b_src, W_dst, b_dst, blk=1000)
    z01 = jnp.concatenate([z_0, z_1.reshape(n, 3 * chan)], axis=1)

    s, zg = _sc_gather(src, dst, px, qx, z01)
    psi0, psi1, psi2, psi3 = _tc_gates(
        s, edgelabels, zg, W_label, b_label, W_gate, b_gate, blk=640)

    zeros = jnp.zeros((n, chan), jnp.float32)
    o0, o1, o2, o3 = _sc_scatter(src, psi0, psi1, psi2, psi3, zeros)
    out0 = o0
    out1 = jnp.stack([o1, o2, o3], axis=1)
    return (out0, out1)


# speedup vs baseline: 17.8860x; 1.2410x over previous
import functools

import jax
import jax.numpy as jnp
from jax import lax
from jax.experimental import pallas as pl
from jax.experimental.pallas import tpu as pltpu
from jax.experimental.pallas import tpu_sc as plsc

NC = 2
NS = 16
NW = NC * NS
CK = 80


def _node_proj_body(emb_ref, pos_ref, wsrc_ref, bsrc_ref, wdst_ref, bdst_ref,
                    p_ref, q_ref):
    chan = emb_ref.shape[1]
    e = emb_ref[...]
    blk = e.shape[0]
    posb = pos_ref[...]
    pad = jnp.zeros((blk, chan - posb.shape[1]), jnp.float32)
    p_ref[...] = jnp.concatenate([
        jnp.dot(e, wsrc_ref[...], preferred_element_type=jnp.float32)
        + bsrc_ref[...], -0.1 * posb, pad], axis=1)
    q_ref[...] = jnp.concatenate([
        jnp.dot(e, wdst_ref[...], preferred_element_type=jnp.float32)
        + bdst_ref[...], 0.1 * posb, pad], axis=1)


def _node_proj(emb, pos, w_src, b_src, w_dst, b_dst, blk):
    n, chan = emb.shape
    pc = pos.shape[1]
    grid = n // blk
    full = lambda i: (0, 0)
    return pl.pallas_call(
        _node_proj_body,
        grid=(grid,),
        in_specs=[
            pl.BlockSpec((blk, chan), lambda i: (i, 0)),
            pl.BlockSpec((blk, pc), lambda i: (i, 0)),
            pl.BlockSpec((chan, chan), full),
            pl.BlockSpec((1, chan), full),
            pl.BlockSpec((chan, chan), full),
            pl.BlockSpec((1, chan), full),
        ],
        out_specs=[
            pl.BlockSpec((blk, 2 * chan), lambda i: (i, 0)),
            pl.BlockSpec((blk, 2 * chan), lambda i: (i, 0)),
        ],
        out_shape=[
            jax.ShapeDtypeStruct((n, 2 * chan), jnp.float32),
            jax.ShapeDtypeStruct((n, 2 * chan), jnp.float32),
        ],
    )(emb, pos, w_src, b_src.reshape(1, chan), w_dst, b_dst.reshape(1, chan))


def _sc_gather_body(src_hbm, dst_hbm, p_hbm, q_hbm, z01_hbm,
                    s_hbm, zg_hbm,
                    srcv, dstv, pgv, qgv, zgv,
                    s0, s1, s2):
    cid = lax.axis_index("c")
    sid = lax.axis_index("s")
    wid = cid * NS + sid
    epw = src_hbm.shape[0] // NW
    base = wid * epw
    nch = epw // CK
    nv = pgv.shape[1] // 16

    def chunk(t, carry):
        off = base + t * CK
        pltpu.sync_copy(src_hbm.at[pl.ds(off, CK)], srcv)
        pltpu.sync_copy(dst_hbm.at[pl.ds(off, CK)], dstv)
        d1 = pltpu.async_copy(p_hbm.at[srcv], pgv, s0)
        d2 = pltpu.async_copy(q_hbm.at[dstv], qgv, s1)
        d3 = pltpu.async_copy(z01_hbm.at[dstv], zgv, s2)
        d1.wait()
        d2.wait()

        def abody(e, c):
            for j in range(nv):
                s = pl.ds(j * 16, 16)
                pgv[e, s] = pgv[e, s] + qgv[e, s]
            return c
        lax.fori_loop(0, CK, abody, 0)

        d3.wait()
        w1 = pltpu.async_copy(pgv, s_hbm.at[pl.ds(off, CK)], s0)
        w2 = pltpu.async_copy(zgv, zg_hbm.at[pl.ds(off, CK)], s1)
        w1.wait()
        w2.wait()
        return carry

    lax.fori_loop(0, nch, chunk, 0)


def _sc_gather(src, dst, p, q, z01):
    e = src.shape[0]
    pw = p.shape[1]
    zc = z01.shape[1]
    mesh = plsc.VectorSubcoreMesh(core_axis_name="c", subcore_axis_name="s",
                                  num_cores=NC, num_subcores=NS)
    f = functools.partial(
        pl.kernel,
        out_type=[
            jax.ShapeDtypeStruct((e, pw), jnp.float32),
            jax.ShapeDtypeStruct((e, zc), jnp.float32),
        ],
        mesh=mesh,
        scratch_types=[
            pltpu.VMEM((CK,), jnp.int32),
            pltpu.VMEM((CK,), jnp.int32),
            pltpu.VMEM((CK, pw), jnp.float32),
            pltpu.VMEM((CK, pw), jnp.float32),
            pltpu.VMEM((CK, zc), jnp.float32),
            pltpu.SemaphoreType.DMA,
            pltpu.SemaphoreType.DMA,
            pltpu.SemaphoreType.DMA,
        ],
    )(_sc_gather_body)
    return f(src, dst, p, q, z01)


def _tc_gate_body(s_ref, lab_ref, zg_ref, wl_ref, bl_ref, wg_ref,
                  bg_ref, p0_ref, p1_ref, p2_ref, p3_ref):
    chan = wl_ref.shape[1]
    sv = s_ref[...]
    a = (sv[:, :chan]
         + jnp.dot(lab_ref[...], wl_ref[...], preferred_element_type=jnp.float32)
         + bl_ref[...])
    r = sv[:, chan:chan + 3]
    s = a * jax.nn.sigmoid(a)
    g = jnp.dot(s, wg_ref[...], preferred_element_type=jnp.float32) + bg_ref[...]
    g0 = g[:, :chan]
    g1 = g[:, chan:2 * chan]
    g2 = g[:, 2 * chan:]
    zg = zg_ref[...]
    p0_ref[...] = g0 * zg[:, :chan]
    p1_ref[...] = g1 * zg[:, chan:2 * chan] + g2 * r[:, 0:1]
    p2_ref[...] = g1 * zg[:, 2 * chan:3 * chan] + g2 * r[:, 1:2]
    p3_ref[...] = g1 * zg[:, 3 * chan:] + g2 * r[:, 2:3]


def _tc_gates(s, lab, zg, w_label, b_label, w_gate, b_gate, blk):
    e, pw = s.shape
    ed, chan = w_label.shape
    zc = zg.shape[1]
    grid = e // blk
    full = lambda i: (0, 0)
    row = lambda i: (i, 0)
    outs = [jax.ShapeDtypeStruct((e, chan), jnp.float32) for _ in range(4)]
    return pl.pallas_call(
        _tc_gate_body,
        grid=(grid,),
        in_specs=[
            pl.BlockSpec((blk, pw), row),
            pl.BlockSpec((blk, ed), row),
            pl.BlockSpec((blk, zc), row),
            pl.BlockSpec((ed, chan), full),
            pl.BlockSpec((1, chan), full),
            pl.BlockSpec((chan, 3 * chan), full),
            pl.BlockSpec((1, 3 * chan), full),
        ],
        out_specs=[pl.BlockSpec((blk, chan), row) for _ in range(4)],
        out_shape=outs,
    )(s, lab, zg, w_label, b_label.reshape(1, chan), w_gate,
      b_gate.reshape(1, 3 * chan))


def _sc_scatter_body(src_hbm, p0_hbm, p1_hbm, p2_hbm, p3_hbm, zeros_hbm,
                     o0_hbm, o1_hbm, o2_hbm, o3_hbm,
                     idx0, idx1, upd0, upd1, acc, lb0, lb1, sb0, sb1):
    cid = lax.axis_index("c")
    sid = lax.axis_index("s")
    e = src_hbm.shape[0]
    n = zeros_hbm.shape[0]
    epc = e // NS
    nch = epc // CK
    nhalf = nch // 2
    rps = (n // NS) & ~7
    tail = n - NS * rps

    def do_group(psi_hbm, out_hbm):
        base = sid * epc

        def fire_load(t, idxv, updv, sem):
            off = base + t * CK
            pltpu.async_copy(src_hbm.at[pl.ds(off, CK)], idxv, sem)
            pltpu.async_copy(psi_hbm.at[pl.ds(off, CK)], updv, sem)

        def wait_load(idxv, updv, sem):
            pltpu.make_async_copy(src_hbm.at[pl.ds(0, CK)], idxv, sem).wait()
            pltpu.make_async_copy(psi_hbm.at[pl.ds(0, CK)], updv, sem).wait()

        def wait_scat(idxv, updv, sem):
            pltpu.make_async_copy(updv, acc.at[idxv], sem).wait()

        rows = pl.ds(sid * rps, rps)
        trows = pl.ds(NS * rps, tail)
        pltpu.sync_copy(zeros_hbm.at[rows], acc.at[rows])

        @pl.when(sid == NS - 1)
        def _():
            pltpu.sync_copy(zeros_hbm.at[trows], acc.at[trows])

        plsc.subcore_barrier()
        fire_load(0, idx0, upd0, lb0)

        def pair(t2, carry):
            ta = 2 * t2

            @pl.when(t2 > 0)
            def _():
                wait_scat(idx1, upd1, sb1)

            fire_load(ta + 1, idx1, upd1, lb1)
            wait_load(idx0, upd0, lb0)
            pltpu.async_copy(upd0, acc.at[idx0], sb0, add=True)

            @pl.when(t2 < nhalf - 1)
            def _():
                wait_scat(idx0, upd0, sb0)
                fire_load(ta + 2, idx0, upd0, lb0)

            wait_load(idx1, upd1, lb1)
            pltpu.async_copy(upd1, acc.at[idx1], sb1, add=True)
            return carry

        lax.fori_loop(0, nhalf, pair, 0)
        wait_scat(idx0, upd0, sb0)
        wait_scat(idx1, upd1, sb1)
        plsc.subcore_barrier()
        pltpu.sync_copy(acc.at[rows], out_hbm.at[rows])

        @pl.when(sid == NS - 1)
        def _():
            pltpu.sync_copy(acc.at[trows], out_hbm.at[trows])

        plsc.subcore_barrier()

    @pl.when(cid == 0)
    def _():
        do_group(p0_hbm, o0_hbm)
        do_group(p1_hbm, o1_hbm)

    @pl.when(cid == 1)
    def _():
        do_group(p2_hbm, o2_hbm)
        do_group(p3_hbm, o3_hbm)


def _sc_scatter(src, p0, p1, p2, p3, zeros):
    n, chan = zeros.shape
    mesh = plsc.VectorSubcoreMesh(core_axis_name="c", subcore_axis_name="s",
                                  num_cores=NC, num_subcores=NS)
    out = jax.ShapeDtypeStruct((n, chan), jnp.float32)
    f = functools.partial(
        pl.kernel,
        out_type=[out, out, out, out],
        mesh=mesh,
        scratch_types=[
            pltpu.VMEM((CK,), jnp.int32),
            pltpu.VMEM((CK,), jnp.int32),
            pltpu.VMEM((CK, chan), jnp.float32),
            pltpu.VMEM((CK, chan), jnp.float32),
            pltpu.VMEM_SHARED((n, chan), jnp.float32),
            pltpu.SemaphoreType.DMA,
            pltpu.SemaphoreType.DMA,
            pltpu.SemaphoreType.DMA,
            pltpu.SemaphoreType.DMA,
        ],
    )(_sc_scatter_body)
    return f(src, p0, p1, p2, p3, zeros)


def kernel(graph, pos, z_0, z_1, emb, edgelabels,
           W_label, b_label, W_src, b_src, W_dst, b_dst, W_gate, b_gate):
    n, chan = z_0.shape
    src = graph[0]
    dst = graph[1]

    px, qx = _node_proj(emb, pos, W_src, b_src, W_dst, b_dst, blk=1000)
    z01 = jnp.concatenate([z_0, z_1.reshape(n, 3 * chan)], axis=1)

    s, zg = _sc_gather(src, dst, px, qx, z01)
    psi0, psi1, psi2, psi3 = _tc_gates(
        s, edgelabels, zg, W_label, b_label, W_gate, b_gate, blk=640)

    zeros = jnp.zeros((n, chan), jnp.float32)
    o0, o1, o2, o3 = _sc_scatter(src, psi0, psi1, psi2, psi3, zeros)
    out0 = o0
    out1 = jnp.stack([o1, o2, o3], axis=1)
    return (out0, out1)


# speedup vs baseline: 19.8993x; 1.1126x over previous
import functools

import jax
import jax.numpy as jnp
from jax import lax
from jax.experimental import pallas as pl
from jax.experimental.pallas import tpu as pltpu
from jax.experimental.pallas import tpu_sc as plsc

NC = 2
NS = 16
NW = NC * NS
CK1 = 40
CK2 = 80


def _node_proj_body(emb_ref, pos_ref, wsrc_ref, bsrc_ref, wdst_ref, bdst_ref,
                    p_ref, q_ref):
    chan = emb_ref.shape[1]
    e = emb_ref[...]
    blk = e.shape[0]
    posb = pos_ref[...]
    pad = jnp.zeros((blk, chan - posb.shape[1]), jnp.float32)
    p_ref[...] = jnp.concatenate([
        jnp.dot(e, wsrc_ref[...], preferred_element_type=jnp.float32)
        + bsrc_ref[...], -0.1 * posb, pad], axis=1)
    q_ref[...] = jnp.concatenate([
        jnp.dot(e, wdst_ref[...], preferred_element_type=jnp.float32)
        + bdst_ref[...], 0.1 * posb, pad], axis=1)


def _node_proj(emb, pos, w_src, b_src, w_dst, b_dst, blk):
    n, chan = emb.shape
    pc = pos.shape[1]
    grid = n // blk
    full = lambda i: (0, 0)
    return pl.pallas_call(
        _node_proj_body,
        grid=(grid,),
        in_specs=[
            pl.BlockSpec((blk, chan), lambda i: (i, 0)),
            pl.BlockSpec((blk, pc), lambda i: (i, 0)),
            pl.BlockSpec((chan, chan), full),
            pl.BlockSpec((1, chan), full),
            pl.BlockSpec((chan, chan), full),
            pl.BlockSpec((1, chan), full),
        ],
        out_specs=[
            pl.BlockSpec((blk, 2 * chan), lambda i: (i, 0)),
            pl.BlockSpec((blk, 2 * chan), lambda i: (i, 0)),
        ],
        out_shape=[
            jax.ShapeDtypeStruct((n, 2 * chan), jnp.float32),
            jax.ShapeDtypeStruct((n, 2 * chan), jnp.float32),
        ],
    )(emb, pos, w_src, b_src.reshape(1, chan), w_dst, b_dst.reshape(1, chan))


def _sc_gather_body(src_hbm, dst_hbm, p_hbm, q_hbm, z01_hbm,
                    s_hbm, zg_hbm,
                    sidx, didx, pg0, pg1, qg0, qg1, zg0, zg1,
                    g0s, g1s, w0s, w1s):
    cid = lax.axis_index("c")
    sid = lax.axis_index("s")
    wid = cid * NS + sid
    epw = src_hbm.shape[0] // NW
    base = wid * epw
    nch = epw // CK1
    nh = nch // 2
    nv = pg0.shape[1] // 16

    pltpu.sync_copy(src_hbm.at[pl.ds(base, epw)], sidx)
    pltpu.sync_copy(dst_hbm.at[pl.ds(base, epw)], didx)

    def fire_g(t, pgv, qgv, zgv, sem):
        ck = pl.ds(t * CK1, CK1)
        pltpu.async_copy(p_hbm.at[sidx.at[ck]], pgv, sem)
        pltpu.async_copy(q_hbm.at[didx.at[ck]], qgv, sem)
        pltpu.async_copy(z01_hbm.at[didx.at[ck]], zgv, sem)

    def wait_g(pgv, qgv, zgv, sem):
        ck = pl.ds(0, CK1)
        pltpu.make_async_copy(p_hbm.at[sidx.at[ck]], pgv, sem).wait()
        pltpu.make_async_copy(q_hbm.at[didx.at[ck]], qgv, sem).wait()
        pltpu.make_async_copy(z01_hbm.at[didx.at[ck]], zgv, sem).wait()

    def add_pq(pgv, qgv):
        def abody(e, c):
            for j in range(nv):
                s = pl.ds(j * 16, 16)
                pgv[e, s] = pgv[e, s] + qgv[e, s]
            return c
        lax.fori_loop(0, CK1, abody, 0)

    def fire_w(t, pgv, zgv, sem):
        off = base + t * CK1
        pltpu.async_copy(pgv, s_hbm.at[pl.ds(off, CK1)], sem)
        pltpu.async_copy(zgv, zg_hbm.at[pl.ds(off, CK1)], sem)

    def wait_w(pgv, zgv, sem):
        pltpu.make_async_copy(pgv, s_hbm.at[pl.ds(0, CK1)], sem).wait()
        pltpu.make_async_copy(zgv, zg_hbm.at[pl.ds(0, CK1)], sem).wait()

    fire_g(0, pg0, qg0, zg0, g0s)

    def pair(t2, carry):
        ta = 2 * t2
        tb = ta + 1

        @pl.when(t2 > 0)
        def _():
            wait_w(pg1, zg1, w1s)

        fire_g(tb, pg1, qg1, zg1, g1s)
        wait_g(pg0, qg0, zg0, g0s)
        add_pq(pg0, qg0)
        fire_w(ta, pg0, zg0, w0s)
        wait_g(pg1, qg1, zg1, g1s)
        add_pq(pg1, qg1)

        @pl.when(t2 < nh - 1)
        def _():
            wait_w(pg0, zg0, w0s)
            fire_g(ta + 2, pg0, qg0, zg0, g0s)

        fire_w(tb, pg1, zg1, w1s)
        return carry

    lax.fori_loop(0, nh, pair, 0)
    wait_w(pg0, zg0, w0s)
    wait_w(pg1, zg1, w1s)


def _sc_gather(src, dst, p, q, z01):
    e = src.shape[0]
    epw = e // NW
    pw = p.shape[1]
    zc = z01.shape[1]
    mesh = plsc.VectorSubcoreMesh(core_axis_name="c", subcore_axis_name="s",
                                  num_cores=NC, num_subcores=NS)
    f = functools.partial(
        pl.kernel,
        out_type=[
            jax.ShapeDtypeStruct((e, pw), jnp.float32),
            jax.ShapeDtypeStruct((e, zc), jnp.float32),
        ],
        mesh=mesh,
        scratch_types=[
            pltpu.VMEM((epw,), jnp.int32),
            pltpu.VMEM((epw,), jnp.int32),
            pltpu.VMEM((CK1, pw), jnp.float32),
            pltpu.VMEM((CK1, pw), jnp.float32),
            pltpu.VMEM((CK1, pw), jnp.float32),
            pltpu.VMEM((CK1, pw), jnp.float32),
            pltpu.VMEM((CK1, zc), jnp.float32),
            pltpu.VMEM((CK1, zc), jnp.float32),
            pltpu.SemaphoreType.DMA,
            pltpu.SemaphoreType.DMA,
            pltpu.SemaphoreType.DMA,
            pltpu.SemaphoreType.DMA,
        ],
    )(_sc_gather_body)
    return f(src, dst, p, q, z01)


def _tc_gate_body(s_ref, lab_ref, zg_ref, wl_ref, bl_ref, wg_ref,
                  bg_ref, p0_ref, p1_ref, p2_ref, p3_ref):
    chan = wl_ref.shape[1]
    sv = s_ref[...]
    a = (sv[:, :chan]
         + jnp.dot(lab_ref[...], wl_ref[...], preferred_element_type=jnp.float32)
         + bl_ref[...])
    r = sv[:, chan:chan + 3]
    s = a * jax.nn.sigmoid(a)
    g = jnp.dot(s, wg_ref[...], preferred_element_type=jnp.float32) + bg_ref[...]
    g0 = g[:, :chan]
    g1 = g[:, chan:2 * chan]
    g2 = g[:, 2 * chan:]
    zg = zg_ref[...]
    p0_ref[...] = g0 * zg[:, :chan]
    p1_ref[...] = g1 * zg[:, chan:2 * chan] + g2 * r[:, 0:1]
    p2_ref[...] = g1 * zg[:, 2 * chan:3 * chan] + g2 * r[:, 1:2]
    p3_ref[...] = g1 * zg[:, 3 * chan:] + g2 * r[:, 2:3]


def _tc_gates(s, lab, zg, w_label, b_label, w_gate, b_gate, blk):
    e, pw = s.shape
    ed, chan = w_label.shape
    zc = zg.shape[1]
    grid = e // blk
    full = lambda i: (0, 0)
    row = lambda i: (i, 0)
    outs = [jax.ShapeDtypeStruct((e, chan), jnp.float32) for _ in range(4)]
    return pl.pallas_call(
        _tc_gate_body,
        grid=(grid,),
        in_specs=[
            pl.BlockSpec((blk, pw), row),
            pl.BlockSpec((blk, ed), row),
            pl.BlockSpec((blk, zc), row),
            pl.BlockSpec((ed, chan), full),
            pl.BlockSpec((1, chan), full),
            pl.BlockSpec((chan, 3 * chan), full),
            pl.BlockSpec((1, 3 * chan), full),
        ],
        out_specs=[pl.BlockSpec((blk, chan), row) for _ in range(4)],
        out_shape=outs,
    )(s, lab, zg, w_label, b_label.reshape(1, chan), w_gate,
      b_gate.reshape(1, 3 * chan))


def _sc_scatter_body(src_hbm, p0_hbm, p1_hbm, p2_hbm, p3_hbm, zeros_hbm,
                     o0_hbm, o1_hbm, o2_hbm, o3_hbm,
                     idx0, idx1, u0, u1, acc, l0s, l1s, s0s, s1s):
    cid = lax.axis_index("c")
    sid = lax.axis_index("s")
    e = src_hbm.shape[0]
    n = zeros_hbm.shape[0]
    epc = e // NS
    nch = epc // CK2
    nh = nch // 2
    rps = (n // NS) & ~7
    tail = n - NS * rps

    def do_group(psi_hbm, out_hbm):
        base = sid * epc

        def fire_lu(t, idxv, uv, sem):
            off = base + t * CK2
            pltpu.async_copy(src_hbm.at[pl.ds(off, CK2)], idxv, sem)
            pltpu.async_copy(psi_hbm.at[pl.ds(off, CK2)], uv, sem)

        def wait_lu(idxv, uv, sem):
            pltpu.make_async_copy(src_hbm.at[pl.ds(0, CK2)], idxv, sem).wait()
            pltpu.make_async_copy(psi_hbm.at[pl.ds(0, CK2)], uv, sem).wait()

        def wait_s(idxv, uv, sem):
            pltpu.make_async_copy(uv, acc.at[idxv], sem).wait()

        rows = pl.ds(sid * rps, rps)
        trows = pl.ds(NS * rps, tail)
        pltpu.sync_copy(zeros_hbm.at[rows], acc.at[rows])

        @pl.when(sid == NS - 1)
        def _():
            pltpu.sync_copy(zeros_hbm.at[trows], acc.at[trows])

        plsc.subcore_barrier()
        fire_lu(0, idx0, u0, l0s)

        def pair(t2, carry):
            ta = 2 * t2

            @pl.when(t2 > 0)
            def _():
                wait_s(idx1, u1, s1s)

            fire_lu(ta + 1, idx1, u1, l1s)
            wait_lu(idx0, u0, l0s)
            pltpu.async_copy(u0, acc.at[idx0], s0s, add=True)

            @pl.when(t2 < nh - 1)
            def _():
                wait_s(idx0, u0, s0s)
                fire_lu(ta + 2, idx0, u0, l0s)

            wait_lu(idx1, u1, l1s)
            pltpu.async_copy(u1, acc.at[idx1], s1s, add=True)
            return carry

        lax.fori_loop(0, nh, pair, 0)
        wait_s(idx0, u0, s0s)
        wait_s(idx1, u1, s1s)
        plsc.subcore_barrier()
        pltpu.sync_copy(acc.at[rows], out_hbm.at[rows])

        @pl.when(sid == NS - 1)
        def _():
            pltpu.sync_copy(acc.at[trows], out_hbm.at[trows])

        plsc.subcore_barrier()

    @pl.when(cid == 0)
    def _():
        do_group(p0_hbm, o0_hbm)
        do_group(p1_hbm, o1_hbm)

    @pl.when(cid == 1)
    def _():
        do_group(p2_hbm, o2_hbm)
        do_group(p3_hbm, o3_hbm)


def _sc_scatter(src, p0, p1, p2, p3, zeros):
    n, chan = zeros.shape
    mesh = plsc.VectorSubcoreMesh(core_axis_name="c", subcore_axis_name="s",
                                  num_cores=NC, num_subcores=NS)
    out = jax.ShapeDtypeStruct((n, chan), jnp.float32)
    f = functools.partial(
        pl.kernel,
        out_type=[out, out, out, out],
        mesh=mesh,
        scratch_types=[
            pltpu.VMEM((CK2,), jnp.int32),
            pltpu.VMEM((CK2,), jnp.int32),
            pltpu.VMEM((CK2, chan), jnp.float32),
            pltpu.VMEM((CK2, chan), jnp.float32),
            pltpu.VMEM_SHARED((n, chan), jnp.float32),
            pltpu.SemaphoreType.DMA,
            pltpu.SemaphoreType.DMA,
            pltpu.SemaphoreType.DMA,
            pltpu.SemaphoreType.DMA,
        ],
    )(_sc_scatter_body)
    return f(src, p0, p1, p2, p3, zeros)


def kernel(graph, pos, z_0, z_1, emb, edgelabels,
           W_label, b_label, W_src, b_src, W_dst, b_dst, W_gate, b_gate):
    n, chan = z_0.shape
    src = graph[0]
    dst = graph[1]

    px, qx = _node_proj(emb, pos, W_src, b_src, W_dst, b_dst, blk=1000)
    z01 = jnp.concatenate([z_0, z_1.reshape(n, 3 * chan)], axis=1)

    s, zg = _sc_gather(src, dst, px, qx, z01)
    psi0, psi1, psi2, psi3 = _tc_gates(
        s, edgelabels, zg, W_label, b_label, W_gate, b_gate, blk=640)

    zeros = jnp.zeros((n, chan), jnp.float32)
    o0, o1, o2, o3 = _sc_scatter(src, psi0, psi1, psi2, psi3, zeros)
    out0 = o0
    out1 = jnp.stack([o1, o2, o3], axis=1)
    return (out0, out1)


# speedup vs baseline: 21.8113x; 1.0961x over previous
import functools

import jax
import jax.numpy as jnp
from jax import lax
from jax.experimental import pallas as pl
from jax.experimental.pallas import tpu as pltpu
from jax.experimental.pallas import tpu_sc as plsc

NC = 2
NS = 16
NW = NC * NS
SPLIT = 2


def _node_proj_body(emb_ref, pos_ref, wsrc_ref, bsrc_ref, wdst_ref, bdst_ref,
                    p_ref, q_ref):
    chan = emb_ref.shape[1]
    e = emb_ref[...]
    blk = e.shape[0]
    posb = pos_ref[...]
    pad = jnp.zeros((blk, chan - posb.shape[1]), jnp.float32)
    p_ref[...] = jnp.concatenate([
        jnp.dot(e, wsrc_ref[...], preferred_element_type=jnp.float32)
        + bsrc_ref[...], -0.1 * posb, pad], axis=1)
    q_ref[...] = jnp.concatenate([
        jnp.dot(e, wdst_ref[...], preferred_element_type=jnp.float32)
        + bdst_ref[...], 0.1 * posb, pad], axis=1)


def _node_proj(emb, pos, w_src, b_src, w_dst, b_dst, blk):
    n, chan = emb.shape
    pc = pos.shape[1]
    grid = n // blk
    full = lambda i: (0, 0)
    return pl.pallas_call(
        _node_proj_body,
        grid=(grid,),
        in_specs=[
            pl.BlockSpec((blk, chan), lambda i: (i, 0)),
            pl.BlockSpec((blk, pc), lambda i: (i, 0)),
            pl.BlockSpec((chan, chan), full),
            pl.BlockSpec((1, chan), full),
            pl.BlockSpec((chan, chan), full),
            pl.BlockSpec((1, chan), full),
        ],
        out_specs=[
            pl.BlockSpec((blk, 2 * chan), lambda i: (i, 0)),
            pl.BlockSpec((blk, 2 * chan), lambda i: (i, 0)),
        ],
        out_shape=[
            jax.ShapeDtypeStruct((n, 2 * chan), jnp.float32),
            jax.ShapeDtypeStruct((n, 2 * chan), jnp.float32),
        ],
    )(emb, pos, w_src, b_src.reshape(1, chan), w_dst, b_dst.reshape(1, chan))


def _make_gather_body(ck):
    def body(src_hbm, dst_hbm, p_hbm, q_hbm, z01_hbm,
             s_hbm, zg_hbm,
             sidx, didx, pg0, pg1, qg0, qg1, zg0, zg1,
             g0s, g1s, w0s, w1s):
        cid = lax.axis_index("c")
        sid = lax.axis_index("s")
        wid = cid * NS + sid
        epw = src_hbm.shape[0] // NW
        base = wid * epw
        nch = epw // ck
        nh = nch // 2
        nv = pg0.shape[1] // 16

        pltpu.sync_copy(src_hbm.at[pl.ds(base, epw)], sidx)
        pltpu.sync_copy(dst_hbm.at[pl.ds(base, epw)], didx)

        def fire_g(t, pgv, qgv, zgv, sem):
            sl = pl.ds(t * ck, ck)
            pltpu.async_copy(p_hbm.at[sidx.at[sl]], pgv, sem)
            pltpu.async_copy(q_hbm.at[didx.at[sl]], qgv, sem)
            pltpu.async_copy(z01_hbm.at[didx.at[sl]], zgv, sem)

        def wait_g(pgv, qgv, zgv, sem):
            sl = pl.ds(0, ck)
            pltpu.make_async_copy(p_hbm.at[sidx.at[sl]], pgv, sem).wait()
            pltpu.make_async_copy(q_hbm.at[didx.at[sl]], qgv, sem).wait()
            pltpu.make_async_copy(z01_hbm.at[didx.at[sl]], zgv, sem).wait()

        def add_pq(pgv, qgv):
            def abody(e, c):
                for j in range(nv):
                    s = pl.ds(j * 16, 16)
                    pgv[e, s] = pgv[e, s] + qgv[e, s]
                return c
            lax.fori_loop(0, ck, abody, 0)

        def fire_w(t, pgv, zgv, sem):
            off = base + t * ck
            pltpu.async_copy(pgv, s_hbm.at[pl.ds(off, ck)], sem)
            pltpu.async_copy(zgv, zg_hbm.at[pl.ds(off, ck)], sem)

        def wait_w(pgv, zgv, sem):
            pltpu.make_async_copy(pgv, s_hbm.at[pl.ds(0, ck)], sem).wait()
            pltpu.make_async_copy(zgv, zg_hbm.at[pl.ds(0, ck)], sem).wait()

        fire_g(0, pg0, qg0, zg0, g0s)

        def pair(t2, carry):
            ta = 2 * t2
            tb = ta + 1

            @pl.when(t2 > 0)
            def _():
                wait_w(pg1, zg1, w1s)

            fire_g(tb, pg1, qg1, zg1, g1s)
            wait_g(pg0, qg0, zg0, g0s)
            add_pq(pg0, qg0)
            fire_w(ta, pg0, zg0, w0s)
            wait_g(pg1, qg1, zg1, g1s)
            add_pq(pg1, qg1)

            @pl.when(ta + 2 < nch)
            def _():
                wait_w(pg0, zg0, w0s)
                fire_g(ta + 2, pg0, qg0, zg0, g0s)

            fire_w(tb, pg1, zg1, w1s)
            return carry

        lax.fori_loop(0, nh, pair, 0)
        if nch % 2:
            wait_g(pg0, qg0, zg0, g0s)
            add_pq(pg0, qg0)
            fire_w(nch - 1, pg0, zg0, w0s)
        wait_w(pg0, zg0, w0s)
        wait_w(pg1, zg1, w1s)

    return body


def _sc_gather(src, dst, p, q, z01, ck):
    e = src.shape[0]
    epw = e // NW
    pw = p.shape[1]
    zc = z01.shape[1]
    mesh = plsc.VectorSubcoreMesh(core_axis_name="c", subcore_axis_name="s",
                                  num_cores=NC, num_subcores=NS)
    f = functools.partial(
        pl.kernel,
        out_type=[
            jax.ShapeDtypeStruct((e, pw), jnp.float32),
            jax.ShapeDtypeStruct((e, zc), jnp.float32),
        ],
        mesh=mesh,
        scratch_types=[
            pltpu.VMEM((epw,), jnp.int32),
            pltpu.VMEM((epw,), jnp.int32),
            pltpu.VMEM((ck, pw), jnp.float32),
            pltpu.VMEM((ck, pw), jnp.float32),
            pltpu.VMEM((ck, pw), jnp.float32),
            pltpu.VMEM((ck, pw), jnp.float32),
            pltpu.VMEM((ck, zc), jnp.float32),
            pltpu.VMEM((ck, zc), jnp.float32),
            pltpu.SemaphoreType.DMA,
            pltpu.SemaphoreType.DMA,
            pltpu.SemaphoreType.DMA,
            pltpu.SemaphoreType.DMA,
        ],
    )(_make_gather_body(ck))
    return f(src, dst, p, q, z01)


def _tc_gate_body(s_ref, lab_ref, zg_ref, wl_ref, bl_ref, wg_ref,
                  bg_ref, p0_ref, p1_ref, p2_ref, p3_ref):
    chan = wl_ref.shape[1]
    sv = s_ref[...]
    a = (sv[:, :chan]
         + jnp.dot(lab_ref[...], wl_ref[...], preferred_element_type=jnp.float32)
         + bl_ref[...])
    r = sv[:, chan:chan + 3]
    s = a * jax.nn.sigmoid(a)
    g = jnp.dot(s, wg_ref[...], preferred_element_type=jnp.float32) + bg_ref[...]
    g0 = g[:, :chan]
    g1 = g[:, chan:2 * chan]
    g2 = g[:, 2 * chan:]
    zg = zg_ref[...]
    p0_ref[...] = g0 * zg[:, :chan]
    p1_ref[...] = g1 * zg[:, chan:2 * chan] + g2 * r[:, 0:1]
    p2_ref[...] = g1 * zg[:, 2 * chan:3 * chan] + g2 * r[:, 1:2]
    p3_ref[...] = g1 * zg[:, 3 * chan:] + g2 * r[:, 2:3]


def _tc_gates(s, lab, zg, w_label, b_label, w_gate, b_gate, blk):
    e, pw = s.shape
    ed, chan = w_label.shape
    zc = zg.shape[1]
    grid = e // blk
    full = lambda i: (0, 0)
    row = lambda i: (i, 0)
    outs = [jax.ShapeDtypeStruct((e, chan), jnp.float32) for _ in range(4)]
    return pl.pallas_call(
        _tc_gate_body,
        grid=(grid,),
        in_specs=[
            pl.BlockSpec((blk, pw), row),
            pl.BlockSpec((blk, ed), row),
            pl.BlockSpec((blk, zc), row),
            pl.BlockSpec((ed, chan), full),
            pl.BlockSpec((1, chan), full),
            pl.BlockSpec((chan, 3 * chan), full),
            pl.BlockSpec((1, 3 * chan), full),
        ],
        out_specs=[pl.BlockSpec((blk, chan), row) for _ in range(4)],
        out_shape=outs,
    )(s, lab, zg, w_label, b_label.reshape(1, chan), w_gate,
      b_gate.reshape(1, 3 * chan))


def _make_scatter_body(ck):
    def body(src_hbm, p0_hbm, p1_hbm, p2_hbm, p3_hbm,
             i0_hbm, i1_hbm, i2_hbm, i3_hbm,
             o0_hbm, o1_hbm, o2_hbm, o3_hbm,
             idx0, idx1, u0, u1, acc, l0s, l1s, s0s, s1s):
        cid = lax.axis_index("c")
        sid = lax.axis_index("s")
        e = src_hbm.shape[0]
        n = i0_hbm.shape[0]
        epc = e // NS
        nch = epc // ck
        nh = nch // 2
        rps = (n // NS) & ~7
        tail = n - NS * rps

        def do_group(psi_hbm, init_hbm, out_hbm):
            base = sid * epc

            def fire_lu(t, idxv, uv, sem):
                off = base + t * ck
                pltpu.async_copy(src_hbm.at[pl.ds(off, ck)], idxv, sem)
                pltpu.async_copy(psi_hbm.at[pl.ds(off, ck)], uv, sem)

            def wait_lu(idxv, uv, sem):
                pltpu.make_async_copy(src_hbm.at[pl.ds(0, ck)], idxv,
                                      sem).wait()
                pltpu.make_async_copy(psi_hbm.at[pl.ds(0, ck)], uv,
                                      sem).wait()

            def wait_s(idxv, uv, sem):
                pltpu.make_async_copy(uv, acc.at[idxv], sem).wait()

            rows = pl.ds(sid * rps, rps)
            trows = pl.ds(NS * rps, tail)
            pltpu.sync_copy(init_hbm.at[rows], acc.at[rows])

            @pl.when(sid == NS - 1)
            def _():
                pltpu.sync_copy(init_hbm.at[trows], acc.at[trows])

            plsc.subcore_barrier()
            fire_lu(0, idx0, u0, l0s)

            def pair(t2, carry):
                ta = 2 * t2

                @pl.when(t2 > 0)
                def _():
                    wait_s(idx1, u1, s1s)

                fire_lu(ta + 1, idx1, u1, l1s)
                wait_lu(idx0, u0, l0s)
                pltpu.async_copy(u0, acc.at[idx0], s0s, add=True)

                @pl.when(ta + 2 < nch)
                def _():
                    wait_s(idx0, u0, s0s)
                    fire_lu(ta + 2, idx0, u0, l0s)

                wait_lu(idx1, u1, l1s)
                pltpu.async_copy(u1, acc.at[idx1], s1s, add=True)
                return carry

            lax.fori_loop(0, nh, pair, 0)
            if nch % 2:
                wait_lu(idx0, u0, l0s)
                pltpu.async_copy(u0, acc.at[idx0], s0s, add=True)
            wait_s(idx0, u0, s0s)
            wait_s(idx1, u1, s1s)
            plsc.subcore_barrier()
            pltpu.sync_copy(acc.at[rows], out_hbm.at[rows])

            @pl.when(sid == NS - 1)
            def _():
                pltpu.sync_copy(acc.at[trows], out_hbm.at[trows])

            plsc.subcore_barrier()

        @pl.when(cid == 0)
        def _():
            do_group(p0_hbm, i0_hbm, o0_hbm)
            do_group(p1_hbm, i1_hbm, o1_hbm)

        @pl.when(cid == 1)
        def _():
            do_group(p2_hbm, i2_hbm, o2_hbm)
            do_group(p3_hbm, i3_hbm, o3_hbm)

    return body


def _sc_scatter(src, psis, inits, ck):
    n, chan = inits[0].shape
    mesh = plsc.VectorSubcoreMesh(core_axis_name="c", subcore_axis_name="s",
                                  num_cores=NC, num_subcores=NS)
    out = jax.ShapeDtypeStruct((n, chan), jnp.float32)
    f = functools.partial(
        pl.kernel,
        out_type=[out, out, out, out],
        mesh=mesh,
        scratch_types=[
            pltpu.VMEM((ck,), jnp.int32),
            pltpu.VMEM((ck,), jnp.int32),
            pltpu.VMEM((ck, chan), jnp.float32),
            pltpu.VMEM((ck, chan), jnp.float32),
            pltpu.VMEM_SHARED((n, chan), jnp.float32),
            pltpu.SemaphoreType.DMA,
            pltpu.SemaphoreType.DMA,
            pltpu.SemaphoreType.DMA,
            pltpu.SemaphoreType.DMA,
        ],
    )(_make_scatter_body(ck))
    return f(src, *psis, *inits)


def kernel(graph, pos, z_0, z_1, emb, edgelabels,
           W_label, b_label, W_src, b_src, W_dst, b_dst, W_gate, b_gate):
    n, chan = z_0.shape
    src = graph[0]
    dst = graph[1]
    e = src.shape[0]
    eh = e // SPLIT

    px, qx = _node_proj(emb, pos, W_src, b_src, W_dst, b_dst, blk=1000)
    z01 = jnp.concatenate([z_0, z_1.reshape(n, 3 * chan)], axis=1)

    zeros = jnp.zeros((n, chan), jnp.float32)
    parts = (zeros, zeros, zeros, zeros)
    ck1 = 40
    ck2 = 80
    for h in range(SPLIT):
        sl = slice(h * eh, (h + 1) * eh)
        s, zg = _sc_gather(src[sl], dst[sl], px, qx, z01, ck=ck1)
        psis = _tc_gates(s, edgelabels[sl], zg, W_label, b_label,
                         W_gate, b_gate, blk=640)
        parts = _sc_scatter(src[sl], psis, parts, ck=ck2)

    o0, o1, o2, o3 = parts
    out0 = o0
    out1 = jnp.stack([o1, o2, o3], axis=1)
    return (out0, out1)
